# baseline (speedup 1.0000x reference)
import functools

import jax
import jax.numpy as jnp
import numpy as np
from jax.experimental import pallas as pl
from jax.experimental.pallas import tpu as pltpu
from jax.experimental.pallas import tpu_sc as plsc

NEG = -1e9
CAP = 1024
INT_MIN = np.int32(-2147483648)


def _sortable(f):
    b = jax.lax.bitcast_convert_type(f, jnp.int32)
    return jnp.where(b < 0, b ^ np.int32(0x7FFFFFFF), b)


def _bisect(keys, weights, limit, strict):
    def body(b, tu):
        cand_u = tu | (np.int32(1) << (31 - b))
        cand_k = cand_u ^ INT_MIN
        lt = keys < cand_k
        sel = jnp.where(lt, 1.0, 0.0) if weights is None else (
            jnp.where(lt, weights, 0.0))
        mass = jnp.sum(sel, axis=1, keepdims=True)
        ok = (mass < limit) if strict else (mass <= limit)
        return jnp.where(ok, cand_u, tu)

    tu = jax.lax.fori_loop(0, 32, body,
                           jnp.zeros(limit.shape, jnp.int32))
    return tu ^ INT_MIN


def _bisect_stable(keys, idx, weights, limit, strict, lo_ref):
    hk = _bisect(keys, weights, limit, strict)
    at = keys == hk
    n_at = jnp.sum(jnp.where(at, 1.0, 0.0), axis=1, keepdims=True)
    lo_ref[...] = jnp.full(limit.shape, 0x1FFFF, jnp.int32)

    @pl.when(jnp.max(n_at) > 1.0)
    def _tie_refine():
        below = jnp.sum(jnp.where(keys < hk, weights, 0.0), axis=1,
                        keepdims=True)

        def body(b, lo):
            cand = lo | (np.int32(1) << (16 - b))
            mass = below + jnp.sum(
                jnp.where(at & (idx < cand), weights, 0.0), axis=1,
                keepdims=True)
            ok = (mass < limit) if strict else (mass <= limit)
            return jnp.where(ok, cand, lo)

        lo_ref[...] = jax.lax.fori_loop(
            0, 17, body, jnp.full(limit.shape, 0, jnp.int32))

    return (keys < hk) | (at & (idx <= lo_ref[...]))


def _body_a(l_ref, tau_ref, topk_ref, typp_ref, avoid_ref, x3_ref, lo_ref):
    r, v = l_ref.shape
    logits = l_ref[...]
    idx = jax.lax.broadcasted_iota(jnp.int32, (r, v), 1)

    d = ~_sortable(logits)
    rank = avoid_ref[...] + 1
    imax = np.int32(0x7FFFFFFF)
    m = jnp.min(d, axis=1, keepdims=True)
    cum = jnp.sum(jnp.where(d <= m, 1, 0), axis=1, keepdims=True)
    thr = m
    done = cum >= rank
    for _ in range(9):
        m = jnp.min(jnp.where(d > m, d, imax), axis=1, keepdims=True)
        cum = cum + jnp.sum(jnp.where(d == m, 1, 0), axis=1, keepdims=True)
        thr = jnp.where(done, thr, m)
        done = done | (cum >= rank)
    x = jnp.where(d < thr, NEG, logits) / tau_ref[...]

    m = jnp.max(x, axis=1, keepdims=True)
    ex = jnp.exp(x - m)
    s = jnp.sum(ex, axis=1, keepdims=True)
    logp = (x - m) - jnp.log(s)
    probs = jnp.exp(logp)
    ent = -jnp.sum(probs * logp, axis=1, keepdims=True)
    shifted = jnp.abs(-logp - ent)
    keep = _bisect_stable(_sortable(shifted), idx, probs, typp_ref[...],
                          strict=True, lo_ref=lo_ref)
    x = jnp.where(keep, x, NEG)

    d2 = ~_sortable(x)
    k = jnp.clip(topk_ref[...], 1, v).astype(jnp.float32)
    tk = _bisect(d2, None, k, strict=True)
    x3_ref[...] = jnp.where(d2 <= tk, x, NEG)


def _sc_compact(b, v):
    info = plsc.get_sparse_core_info()
    nw = info.num_cores * info.num_subcores
    rows_per_w = b // nw
    mesh = plsc.VectorSubcoreMesh(core_axis_name="c", subcore_axis_name="s")

    @functools.partial(
        pl.kernel, mesh=mesh,
        out_type=[jax.ShapeDtypeStruct((b * CAP,), jnp.float32),
                  jax.ShapeDtypeStruct((b * CAP,), jnp.int32)],
        compiler_params=pltpu.CompilerParams(needs_layout_passes=False),
        scratch_types=[pltpu.VMEM((v,), jnp.float32),
                       pltpu.VMEM((CAP,), jnp.float32),
                       pltpu.VMEM((CAP,), jnp.int32),
                       pltpu.SemaphoreType.DMA],
    )
    def kfn(x3_hbm, vals_hbm, inds_hbm, rowbuf, vbuf, ibuf, dsem):
        wid = jax.lax.axis_index("s") * info.num_cores + \
            jax.lax.axis_index("c")
        lanes = jax.lax.iota(jnp.int32, 16)

        for rr in range(rows_per_w):
            row = wid * rows_per_w + rr
            chunk = v // 4
            copies = [
                pltpu.async_copy(
                    x3_hbm.at[pl.ds(row * v + c * chunk, chunk)],
                    rowbuf.at[pl.ds(c * chunk, chunk)], dsem)
                for c in range(4)
            ]
            for cp in copies:
                cp.wait()

            def init(i, _):
                vbuf[pl.ds(i * 16, 16)] = jnp.full((16,), NEG, jnp.float32)
                ibuf[pl.ds(i * 16, 16)] = jnp.zeros((16,), jnp.int32)
                return 0

            jax.lax.fori_loop(0, CAP // 16, init, 0)

            def step(jj, pos):
                for u in range(4):
                    j = jj * 4 + u
                    vals = rowbuf[pl.ds(j * 16, 16)]
                    msk = vals > -1e6
                    mi = jnp.where(msk, 1, 0)
                    dst = jnp.minimum(pos + plsc.cumsum(mi) - mi, CAP - 1)
                    plsc.store_scatter(vbuf, [dst], vals, mask=msk)
                    plsc.store_scatter(ibuf, [dst], lanes + j * 16,
                                       mask=msk)
                    pos = pos + plsc.all_reduce_population_count(msk)
                return pos

            jax.lax.fori_loop(0, v // 64, step,
                              jnp.zeros((16,), jnp.int32))
            pltpu.sync_copy(vbuf, vals_hbm.at[pl.ds(row * CAP, CAP)])
            pltpu.sync_copy(ibuf, inds_hbm.at[pl.ds(row * CAP, CAP)])

    return kfn


def _compact(x3):
    b, v = x3.shape
    vals, inds = _sc_compact(b, v)(x3.reshape(-1))
    return vals.reshape(b, CAP), inds.reshape(b, CAP)


def _rotl(x, d):
    return (x << np.uint32(d)) | (x >> np.uint32(32 - d))


def _gumbel_at(flat_idx):
    x0 = jnp.zeros(flat_idx.shape, jnp.uint32)
    x1 = flat_idx.astype(jnp.uint32) + np.uint32(1)
    ks = (np.uint32(0), np.uint32(1), np.uint32(0x1BD11BDB))
    rot = ((13, 15, 26, 6), (17, 29, 16, 24))
    for i in range(5):
        for r in rot[i % 2]:
            x0 = x0 + x1
            x1 = x0 ^ _rotl(x1, r)
        x0 = x0 + ks[(i + 1) % 3]
        x1 = x1 + ks[(i + 2) % 3] + np.uint32(i + 1)
    bits = x0 ^ x1
    fb = (bits >> np.uint32(9)) | np.uint32(0x3F800000)
    f = jax.lax.bitcast_convert_type(fb, jnp.float32) - np.float32(1.0)
    tiny = np.float32(np.finfo(np.float32).tiny)
    u = jnp.maximum(tiny, f * (np.float32(1.0) - tiny) + tiny)
    return -jnp.log(-jnp.log(u))


def _body_c(vals_ref, inds_ref, topp_ref, minp_ref, out_ref, lo_ref, v):
    x = vals_ref[...]
    idx = inds_ref[...]

    m3 = jnp.max(x, axis=1, keepdims=True)
    ex3 = jnp.exp(x - m3)
    p3 = ex3 / jnp.sum(ex3, axis=1, keepdims=True)
    keep = _bisect_stable(_sortable(-x), idx, p3, topp_ref[...],
                          strict=False, lo_ref=lo_ref)
    x = jnp.where(keep, x, NEG)

    m4 = jnp.max(x, axis=1, keepdims=True)
    ex4 = jnp.exp(x - m4)
    p4 = ex4 / jnp.sum(ex4, axis=1, keepdims=True)
    maxp = jnp.max(p4, axis=1, keepdims=True)
    x = jnp.where(p4 < minp_ref[...] * maxp, NEG, x)

    rows = jax.lax.broadcasted_iota(jnp.int32, x.shape, 0)
    g = _gumbel_at(rows * v + idx)
    sg = x + g
    mx = jnp.max(sg, axis=1, keepdims=True)
    out_ref[...] = jnp.min(jnp.where(sg == mx, idx, v), axis=1,
                           keepdims=True)


@jax.jit
def _run(logits, temperatures, top_ks, top_ps, min_ps, typical_ps,
         avoid_top_ks):
    b, v = logits.shape
    rows = 16
    col = lambda a, dt: a.reshape(b, 1).astype(dt)
    rspec = pl.BlockSpec((rows, v), lambda i: (i, 0))
    pspec = pl.BlockSpec((rows, 1), lambda i: (i, 0))
    x3 = pl.pallas_call(
        _body_a,
        grid=(b // rows,),
        in_specs=[rspec, pspec, pspec, pspec, pspec],
        out_specs=rspec,
        out_shape=jax.ShapeDtypeStruct((b, v), jnp.float32),
        scratch_shapes=[pltpu.VMEM((rows, 1), jnp.int32)],
        compiler_params=pltpu.CompilerParams(
            dimension_semantics=("parallel",)),
    )(logits, col(temperatures, jnp.float32), col(top_ks, jnp.int32),
      col(typical_ps, jnp.float32), col(avoid_top_ks, jnp.int32))

    vals, inds = _compact(x3)

    tokens = pl.pallas_call(
        functools.partial(_body_c, v=v),
        grid=(1,),
        in_specs=[pl.BlockSpec((b, CAP), lambda i: (0, 0)),
                  pl.BlockSpec((b, CAP), lambda i: (0, 0)),
                  pl.BlockSpec((b, 1), lambda i: (0, 0)),
                  pl.BlockSpec((b, 1), lambda i: (0, 0))],
        out_specs=pl.BlockSpec((b, 1), lambda i: (0, 0)),
        out_shape=jax.ShapeDtypeStruct((b, 1), jnp.int32),
        scratch_shapes=[pltpu.VMEM((b, 1), jnp.int32)],
    )(vals, inds, col(top_ps, jnp.float32), col(min_ps, jnp.float32))
    return tokens.reshape(b)


def kernel(logits, temperatures, top_ks, top_ps, min_ps, typical_ps,
           avoid_top_ks):
    return _run(logits, temperatures, top_ks, top_ps, min_ps, typical_ps,
                avoid_top_ks)


# speedup vs baseline: 1.0591x; 1.0591x over previous
import functools

import jax
import jax.numpy as jnp
import numpy as np
from jax.experimental import pallas as pl
from jax.experimental.pallas import tpu as pltpu
from jax.experimental.pallas import tpu_sc as plsc

NEG = -1e9
CAP = 1024
INT_MIN = np.int32(-2147483648)


def _sortable(f):
    b = jax.lax.bitcast_convert_type(f, jnp.int32)
    return jnp.where(b < 0, b ^ np.int32(0x7FFFFFFF), b)


def _bisect(keys, weights, limit, strict):
    def body(b, tu):
        cand_u = tu | (np.int32(1) << (31 - b))
        cand_k = cand_u ^ INT_MIN
        lt = keys < cand_k
        sel = jnp.where(lt, 1.0, 0.0) if weights is None else (
            jnp.where(lt, weights, 0.0))
        mass = jnp.sum(sel, axis=1, keepdims=True)
        ok = (mass < limit) if strict else (mass <= limit)
        return jnp.where(ok, cand_u, tu)

    tu = jax.lax.fori_loop(0, 32, body,
                           jnp.zeros(limit.shape, jnp.int32))
    return tu ^ INT_MIN


def _bisect_stable(keys, idx, weights, limit, strict, lo_ref):
    hk = _bisect(keys, weights, limit, strict)
    at = keys == hk
    n_at = jnp.sum(jnp.where(at, 1.0, 0.0), axis=1, keepdims=True)
    lo_ref[...] = jnp.full(limit.shape, 0x1FFFF, jnp.int32)

    @pl.when(jnp.max(n_at) > 1.0)
    def _tie_refine():
        below = jnp.sum(jnp.where(keys < hk, weights, 0.0), axis=1,
                        keepdims=True)

        def body(b, lo):
            cand = lo | (np.int32(1) << (16 - b))
            mass = below + jnp.sum(
                jnp.where(at & (idx < cand), weights, 0.0), axis=1,
                keepdims=True)
            ok = (mass < limit) if strict else (mass <= limit)
            return jnp.where(ok, cand, lo)

        lo_ref[...] = jax.lax.fori_loop(
            0, 17, body, jnp.full(limit.shape, 0, jnp.int32))

    return (keys < hk) | (at & (idx <= lo_ref[...]))


def _body_a(l_ref, tau_ref, topk_ref, typp_ref, avoid_ref, x3_ref, lo_ref):
    r, v = l_ref.shape
    logits = l_ref[...]
    idx = jax.lax.broadcasted_iota(jnp.int32, (r, v), 1)

    d = ~_sortable(logits)
    rank = avoid_ref[...] + 1
    imax = np.int32(0x7FFFFFFF)
    m = jnp.min(d, axis=1, keepdims=True)
    cum = jnp.sum(jnp.where(d <= m, 1, 0), axis=1, keepdims=True)
    thr = m
    done = cum >= rank
    for _ in range(9):
        m = jnp.min(jnp.where(d > m, d, imax), axis=1, keepdims=True)
        cum = cum + jnp.sum(jnp.where(d == m, 1, 0), axis=1, keepdims=True)
        thr = jnp.where(done, thr, m)
        done = done | (cum >= rank)
    x = jnp.where(d < thr, NEG, logits) / tau_ref[...]

    m = jnp.max(x, axis=1, keepdims=True)
    ex = jnp.exp(x - m)
    s = jnp.sum(ex, axis=1, keepdims=True)
    logp = (x - m) - jnp.log(s)
    probs = jnp.exp(logp)
    ent = -jnp.sum(probs * logp, axis=1, keepdims=True)
    shifted = jnp.abs(-logp - ent)
    keep = _bisect_stable(_sortable(shifted), idx, probs, typp_ref[...],
                          strict=True, lo_ref=lo_ref)
    x = jnp.where(keep, x, NEG)

    d2 = ~_sortable(x)
    k = jnp.clip(topk_ref[...], 1, v).astype(jnp.float32)
    tk = _bisect(d2, None, k, strict=True)
    x3_ref[...] = jnp.where(d2 <= tk, x, NEG)


def _sc_compact(b, v):
    info = plsc.get_sparse_core_info()
    nw = info.num_cores * info.num_subcores
    rows_per_w = b // nw
    mesh = plsc.VectorSubcoreMesh(core_axis_name="c", subcore_axis_name="s")

    @functools.partial(
        pl.kernel, mesh=mesh,
        out_type=[jax.ShapeDtypeStruct((b, CAP), jnp.float32),
                  jax.ShapeDtypeStruct((b, CAP), jnp.int32)],
        compiler_params=pltpu.CompilerParams(needs_layout_passes=False),
        scratch_types=[pltpu.VMEM((v,), jnp.float32),
                       pltpu.VMEM((CAP,), jnp.float32),
                       pltpu.VMEM((CAP,), jnp.int32)],
    )
    def kfn(x3_hbm, vals_hbm, inds_hbm, rowbuf, vbuf, ibuf):
        wid = jax.lax.axis_index("s") * info.num_cores + \
            jax.lax.axis_index("c")
        lanes = jax.lax.iota(jnp.int32, 16)

        for rr in range(rows_per_w):
            row = wid * rows_per_w + rr
            pltpu.sync_copy(x3_hbm.at[row], rowbuf)

            def init(i, _):
                vbuf[pl.ds(i * 16, 16)] = jnp.full((16,), NEG, jnp.float32)
                ibuf[pl.ds(i * 16, 16)] = jnp.zeros((16,), jnp.int32)
                return 0

            jax.lax.fori_loop(0, CAP // 16, init, 0)

            def step(jj, pos):
                for u in range(4):
                    j = jj * 4 + u
                    vals = rowbuf[pl.ds(j * 16, 16)]
                    msk = vals > -1e6
                    mi = jnp.where(msk, 1, 0)
                    dst = jnp.minimum(pos + plsc.cumsum(mi) - mi, CAP - 1)
                    plsc.store_scatter(vbuf, [dst], vals, mask=msk)
                    plsc.store_scatter(ibuf, [dst], lanes + j * 16,
                                       mask=msk)
                    pos = pos + plsc.all_reduce_population_count(msk)
                return pos

            jax.lax.fori_loop(0, v // 64, step,
                              jnp.zeros((16,), jnp.int32))
            pltpu.sync_copy(vbuf, vals_hbm.at[row])
            pltpu.sync_copy(ibuf, inds_hbm.at[row])

    return kfn


def _compact(x3):
    b, v = x3.shape
    return _sc_compact(b, v)(x3)


def _rotl(x, d):
    return (x << np.uint32(d)) | (x >> np.uint32(32 - d))


def _gumbel_at(flat_idx):
    x0 = jnp.zeros(flat_idx.shape, jnp.uint32)
    x1 = flat_idx.astype(jnp.uint32) + np.uint32(1)
    ks = (np.uint32(0), np.uint32(1), np.uint32(0x1BD11BDB))
    rot = ((13, 15, 26, 6), (17, 29, 16, 24))
    for i in range(5):
        for r in rot[i % 2]:
            x0 = x0 + x1
            x1 = x0 ^ _rotl(x1, r)
        x0 = x0 + ks[(i + 1) % 3]
        x1 = x1 + ks[(i + 2) % 3] + np.uint32(i + 1)
    bits = x0 ^ x1
    fb = (bits >> np.uint32(9)) | np.uint32(0x3F800000)
    f = jax.lax.bitcast_convert_type(fb, jnp.float32) - np.float32(1.0)
    tiny = np.float32(np.finfo(np.float32).tiny)
    u = jnp.maximum(tiny, f * (np.float32(1.0) - tiny) + tiny)
    return -jnp.log(-jnp.log(u))


def _body_c(vals_ref, inds_ref, topp_ref, minp_ref, out_ref, lo_ref, v):
    x = vals_ref[...]
    idx = inds_ref[...]

    m3 = jnp.max(x, axis=1, keepdims=True)
    ex3 = jnp.exp(x - m3)
    p3 = ex3 / jnp.sum(ex3, axis=1, keepdims=True)
    keep = _bisect_stable(_sortable(-x), idx, p3, topp_ref[...],
                          strict=False, lo_ref=lo_ref)
    x = jnp.where(keep, x, NEG)

    m4 = jnp.max(x, axis=1, keepdims=True)
    ex4 = jnp.exp(x - m4)
    p4 = ex4 / jnp.sum(ex4, axis=1, keepdims=True)
    maxp = jnp.max(p4, axis=1, keepdims=True)
    x = jnp.where(p4 < minp_ref[...] * maxp, NEG, x)

    rows = jax.lax.broadcasted_iota(jnp.int32, x.shape, 0)
    g = _gumbel_at(rows * v + idx)
    sg = x + g
    mx = jnp.max(sg, axis=1, keepdims=True)
    out_ref[...] = jnp.min(jnp.where(sg == mx, idx, v), axis=1,
                           keepdims=True)


@jax.jit
def _run(logits, temperatures, top_ks, top_ps, min_ps, typical_ps,
         avoid_top_ks):
    b, v = logits.shape
    rows = 16
    col = lambda a, dt: a.reshape(b, 1).astype(dt)
    rspec = pl.BlockSpec((rows, v), lambda i: (i, 0))
    pspec = pl.BlockSpec((rows, 1), lambda i: (i, 0))
    x3 = pl.pallas_call(
        _body_a,
        grid=(b // rows,),
        in_specs=[rspec, pspec, pspec, pspec, pspec],
        out_specs=rspec,
        out_shape=jax.ShapeDtypeStruct((b, v), jnp.float32),
        scratch_shapes=[pltpu.VMEM((rows, 1), jnp.int32)],
        compiler_params=pltpu.CompilerParams(
            dimension_semantics=("parallel",)),
    )(logits, col(temperatures, jnp.float32), col(top_ks, jnp.int32),
      col(typical_ps, jnp.float32), col(avoid_top_ks, jnp.int32))

    vals, inds = _compact(x3)

    tokens = pl.pallas_call(
        functools.partial(_body_c, v=v),
        grid=(1,),
        in_specs=[pl.BlockSpec((b, CAP), lambda i: (0, 0)),
                  pl.BlockSpec((b, CAP), lambda i: (0, 0)),
                  pl.BlockSpec((b, 1), lambda i: (0, 0)),
                  pl.BlockSpec((b, 1), lambda i: (0, 0))],
        out_specs=pl.BlockSpec((b, 1), lambda i: (0, 0)),
        out_shape=jax.ShapeDtypeStruct((b, 1), jnp.int32),
        scratch_shapes=[pltpu.VMEM((b, 1), jnp.int32)],
    )(vals, inds, col(top_ps, jnp.float32), col(min_ps, jnp.float32))
    return tokens.reshape(b)


def kernel(logits, temperatures, top_ks, top_ps, min_ps, typical_ps,
           avoid_top_ks):
    return _run(logits, temperatures, top_ks, top_ps, min_ps, typical_ps,
                avoid_top_ks)
